# initial kernel scaffold (unmeasured)
import jax
import jax.numpy as jnp
from jax import lax
from jax.experimental import pallas as pl
from jax.experimental.pallas import tpu as pltpu


def kernel(
    x,
):
    def body(*refs):
        pass

    out_shape = jax.ShapeDtypeStruct(..., jnp.float32)
    return pl.pallas_call(body, out_shape=out_shape)(...)


# baseline (device time: 55297 ns/iter reference)
import jax
import jax.numpy as jnp
from jax import lax
from jax.experimental import pallas as pl
from jax.experimental.pallas import tpu as pltpu


def kernel(x):
    m, n = x.shape

    def body(x_ref, out_ref, row_send, row_recv, col_send, col_recv, sems):
        my_x = lax.axis_index("x")
        my_y = lax.axis_index("y")

        row_send[:, :] = jnp.where(my_x == 0, x_ref[m - 1:m, :], x_ref[0:1, :])
        col_send[:, :] = jnp.where(my_y == 0, x_ref[:, n - 1:n], x_ref[:, 0:1])

        rdma_row = pltpu.make_async_remote_copy(
            src_ref=row_send,
            dst_ref=row_recv,
            send_sem=sems.at[0],
            recv_sem=sems.at[1],
            device_id=(1 - my_x, my_y),
            device_id_type=pl.DeviceIdType.MESH,
        )
        rdma_col = pltpu.make_async_remote_copy(
            src_ref=col_send,
            dst_ref=col_recv,
            send_sem=sems.at[2],
            recv_sem=sems.at[3],
            device_id=(my_x, 1 - my_y),
            device_id_type=pl.DeviceIdType.MESH,
        )
        rdma_row.start()
        rdma_col.start()

        c = x_ref[:, :]
        out_ref[:, :] = 0.5 * c

        rdma_row.wait()
        rdma_col.wait()

        halo_row = row_recv[:, :]
        halo_col = col_recv[:, :]

        out_ref[:, :] += 0.125 * jnp.concatenate([halo_row, c[:-1, :]], axis=0)
        out_ref[:, :] += 0.125 * jnp.concatenate([c[1:, :], halo_row], axis=0)
        out_ref[:, :] += 0.125 * jnp.concatenate([halo_col, c[:, :-1]], axis=1)
        out_ref[:, :] += 0.125 * jnp.concatenate([c[:, 1:], halo_col], axis=1)

        @pl.when(my_x == 0)
        def _():
            out_ref[0:1, :] = x_ref[0:1, :]

        @pl.when(my_x == 1)
        def _():
            out_ref[m - 1:m, :] = x_ref[m - 1:m, :]

        @pl.when(my_y == 0)
        def _():
            out_ref[:, 0:1] = x_ref[:, 0:1]

        @pl.when(my_y == 1)
        def _():
            out_ref[:, n - 1:n] = x_ref[:, n - 1:n]

    return pl.pallas_call(
        body,
        out_shape=jax.ShapeDtypeStruct((m, n), x.dtype),
        in_specs=[pl.BlockSpec(memory_space=pltpu.VMEM)],
        out_specs=pl.BlockSpec(memory_space=pltpu.VMEM),
        scratch_shapes=[
            pltpu.VMEM((1, n), x.dtype),
            pltpu.VMEM((1, n), x.dtype),
            pltpu.VMEM((m, 1), x.dtype),
            pltpu.VMEM((m, 1), x.dtype),
            pltpu.SemaphoreType.DMA((4,)),
        ],
        compiler_params=pltpu.CompilerParams(
            vmem_limit_bytes=100 * 1024 * 1024,
        ),
    )(x)
